# baseline (speedup 1.0000x reference)
import functools

import jax
import jax.numpy as jnp
from jax.experimental import pallas as pl


def _chamfer_body(km_ref, pn_ref, out_ref, *, n, m, block_n):
    tt = km_ref[:, n:]
    tn = tt[0:1, :] * tt[0:1, :] + tt[1:2, :] * tt[1:2, :] \
        + tt[2:3, :] * tt[2:3, :]

    n_chunks = 8
    mc = m // n_chunks
    g = block_n // 8

    def body(i, carry):
        col8, row_sum = carry
        pn = pn_ref[pl.ds(i * block_n, block_n), :]
        pblk = -2.0 * km_ref[:, pl.ds(i * block_n, block_n)]
        rm_acc = None
        col_parts = []
        for c in range(n_chunks):
            cross = jax.lax.dot_general(
                pblk, tt[:, c * mc:(c + 1) * mc],
                (((0,), (0,)), ((), ())),
                preferred_element_type=jnp.float32)
            d2 = (pn + tn[:, c * mc:(c + 1) * mc]) + cross
            rm_acc = d2 if rm_acc is None else jnp.minimum(rm_acc, d2)
            col_parts.append(jnp.min(d2.reshape(g, 8, mc), axis=0))
        row_min = jnp.min(rm_acc, axis=1, keepdims=True)
        row_sum = row_sum + jnp.sum(
            jnp.sqrt(jnp.maximum(row_min, 0.0) + 1e-12))
        col8 = jnp.minimum(col8, jnp.concatenate(col_parts, axis=1))
        return col8, row_sum

    col8, row_sum = jax.lax.fori_loop(
        0, n // block_n, body,
        (jnp.full((8, m), jnp.inf, dtype=jnp.float32),
         jnp.zeros((1, 1), dtype=jnp.float32)))
    col_min = jnp.min(col8, axis=0, keepdims=True)
    back = jnp.sum(jnp.sqrt(jnp.maximum(col_min, 0.0) + 1e-12),
                   axis=1, keepdims=True)
    out_ref[...] = (row_sum / n + back / m) * 0.5


def kernel(pred, target):
    pred = pred.astype(jnp.float32)
    target = target.astype(jnp.float32)
    n, k = pred.shape
    m, _ = target.shape
    km = jnp.pad(jnp.concatenate([pred, target], axis=0).T,
                 ((0, 8 - k), (0, 0)))
    pn_col = jnp.sum(pred * pred, axis=1, keepdims=True)
    out = pl.pallas_call(
        functools.partial(_chamfer_body, n=n, m=m, block_n=512),
        out_shape=jax.ShapeDtypeStruct((1, 1), jnp.float32),
    )(km, pn_col)
    return out[0, 0]
